# Initial kernel scaffold; baseline (speedup 1.0000x reference)
import jax
import jax.numpy as jnp
from jax.experimental import pallas as pl


def kernel(x, edge_index, edge_weight, W, b):
    raise NotImplementedError("write your pallas kernel here")


# speedup vs baseline: 4.2035x; 4.2035x over previous
import functools

import jax
import jax.numpy as jnp
from jax import lax
from jax.experimental import pallas as pl
from jax.experimental.pallas import tpu as pltpu
from jax.experimental.pallas import tpu_sc as plsc

N = 10000
E = 320000
F = 128
H = 128

NC = 2
NS = 16
NP = 10240
EPC = E // NC
EPT = EPC // NS
CH = 80
NCHUNK = EPT // CH
RP = NP // NS
LANES = 16
FG = F // LANES


def _sc_agg(x, src, dst, w):
    mesh = plsc.VectorSubcoreMesh(core_axis_name="c", subcore_axis_name="s")

    @functools.partial(
        pl.kernel,
        out_type=jax.ShapeDtypeStruct((NC, NP, F), jnp.float32),
        mesh=mesh,
        scratch_types=[
            pltpu.VMEM_SHARED((NP, F), jnp.float32),
            pltpu.VMEM((CH,), jnp.int32),
            pltpu.VMEM((CH,), jnp.int32),
            pltpu.VMEM((CH,), jnp.float32),
            pltpu.VMEM((CH, F), jnp.float32),
            pltpu.SemaphoreType.DMA,
        ],
    )
    def k(x_hbm, src_hbm, dst_hbm, w_hbm, out_hbm, acc, src_v, dst_v, w_v,
          rows_v, sem):
        c = lax.axis_index("c")
        s = lax.axis_index("s")

        zero = jnp.zeros((LANES,), jnp.float32)
        for e in range(CH):
            for f in range(FG):
                rows_v[e, pl.ds(f * LANES, LANES)] = zero

        arow = s * RP
        for j in range(RP // CH):
            pltpu.sync_copy(rows_v, acc.at[pl.ds(arow + j * CH, CH)])
        plsc.subcore_barrier()

        ebase = c * EPC + s * EPT

        def body(i, carry):
            base = ebase + i * CH
            pltpu.sync_copy(src_hbm.at[pl.ds(base, CH)], src_v)
            pltpu.sync_copy(dst_hbm.at[pl.ds(base, CH)], dst_v)
            pltpu.sync_copy(w_hbm.at[pl.ds(base, CH)], w_v)
            pltpu.async_copy(x_hbm.at[src_v], rows_v, sem).wait()
            for g in range(CH // LANES):
                w16 = w_v[pl.ds(g * LANES, LANES)]
                for em in range(LANES):
                    e = g * LANES + em
                    we = jnp.broadcast_to(w16[em:em + 1], (LANES,))
                    for f in range(FG):
                        sl = pl.ds(f * LANES, LANES)
                        rows_v[e, sl] = rows_v[e, sl] * we
            pltpu.sync_copy(rows_v, acc.at[dst_v], add=True)
            return carry

        lax.fori_loop(0, NCHUNK, body, 0)
        plsc.subcore_barrier()

        for j in range(RP // CH):
            pltpu.sync_copy(acc.at[pl.ds(arow + j * CH, CH)], rows_v)
            pltpu.sync_copy(rows_v, out_hbm.at[c, pl.ds(arow + j * CH, CH)])

    return k(x, src, dst, w)


def _combine(p, W, b):
    BR = 1000

    def body(p0_ref, p1_ref, w_ref, b_ref, o_ref):
        acc = p0_ref[0] + p1_ref[0]
        o_ref[...] = (
            jnp.dot(acc, w_ref[...], preferred_element_type=jnp.float32)
            + b_ref[...]
        )

    return pl.pallas_call(
        body,
        grid=(N // BR,),
        in_specs=[
            pl.BlockSpec((1, BR, F), lambda i: (0, i, 0)),
            pl.BlockSpec((1, BR, F), lambda i: (1, i, 0)),
            pl.BlockSpec((F, H), lambda i: (0, 0)),
            pl.BlockSpec((1, H), lambda i: (0, 0)),
        ],
        out_specs=pl.BlockSpec((BR, H), lambda i: (i, 0)),
        out_shape=jax.ShapeDtypeStruct((N, H), jnp.float32),
    )(p, p, W, b.reshape(1, H))


def kernel(x, edge_index, edge_weight, W, b):
    src = edge_index[0].astype(jnp.int32)
    dst = edge_index[1].astype(jnp.int32)
    p = _sc_agg(x, src, dst, edge_weight)
    return _combine(p, W, b)
